# baseline (speedup 1.0000x reference)
import jax
import jax.numpy as jnp
from jax.experimental import pallas as pl
from jax.experimental.pallas import tpu as pltpu

_NB = 8


def _sage_fused(adj_hbm, x_ref,
                wl0_ref, bl0_ref, wr0_ref,
                wl1_ref, bl1_ref, wr1_ref,
                wl2_ref, bl2_ref, wr2_ref,
                out_ref, adj_vmem, sems):
    n = adj_vmem.shape[0]
    w = n // _NB
    copies = [
        pltpu.make_async_copy(adj_hbm.at[pl.ds(b * w, w), :],
                              adj_vmem.at[pl.ds(b * w, w), :],
                              sems.at[b])
        for b in range(_NB)
    ]
    for c in copies:
        c.start()
    g = jnp.transpose(x_ref[...])
    for c in copies:
        c.wait()
    adj = adj_vmem[...]

    deg = jnp.sum(adj, axis=0, keepdims=True)
    dinv = 1.0 / jnp.maximum(deg, 1.0)

    def layer(gh, wl_ref, bl_ref, wr_ref):
        aggT = jnp.dot(gh, adj, preferred_element_type=jnp.float32)
        meanT = aggT * dinv
        lin_l = jnp.dot(jnp.transpose(wl_ref[...]), meanT,
                        preferred_element_type=jnp.float32)
        lin_r = jnp.dot(jnp.transpose(wr_ref[...]), gh,
                        preferred_element_type=jnp.float32)
        return lin_l + lin_r + jnp.transpose(bl_ref[...])

    g = jnp.maximum(layer(g, wl0_ref, bl0_ref, wr0_ref), 0.0)
    g = jnp.maximum(layer(g, wl1_ref, bl1_ref, wr1_ref), 0.0)
    out_ref[...] = jnp.transpose(layer(g, wl2_ref, bl2_ref, wr2_ref))


def kernel(x, adj, W_l0, b_l0, W_r0, W_l1, b_l1, W_r1, W_l2, b_l2, W_r2):
    n, _ = x.shape
    d_out = W_l2.shape[1]
    return pl.pallas_call(
        _sage_fused,
        out_shape=jax.ShapeDtypeStruct((n, d_out), jnp.float32),
        in_specs=[pl.BlockSpec(memory_space=pl.ANY)]
        + [pl.BlockSpec(memory_space=pltpu.VMEM)] * 10,
        scratch_shapes=[pltpu.VMEM((n, n), jnp.float32),
                        pltpu.SemaphoreType.DMA((_NB,))],
    )(adj, x,
      W_l0, b_l0.reshape(1, -1), W_r0,
      W_l1, b_l1.reshape(1, -1), W_r1,
      W_l2, b_l2.reshape(1, -1), W_r2)
